# Initial kernel scaffold; baseline (speedup 1.0000x reference)
import jax
import jax.numpy as jnp
from jax import lax
from jax.experimental import pallas as pl
from jax.experimental.pallas import tpu as pltpu
from jax.experimental.pallas import tpu_sc as plsc

RES = 128
N_RAYS = 65536
N_SAMPLES = 128
NUM_CORES = 2
NUM_SUBCORES = 16
NW = NUM_CORES * NUM_SUBCORES
RPW = N_RAYS // NW
CHUNK = 16
NCHUNK = RPW // CHUNK
NWORDS = RES ** 3 // 32
WPS = NWORDS // NUM_SUBCORES
PACK_CH = 512
INV_N = 1.0 / N_SAMPLES


def _body(rays_hbm, occs_hbm, out_hbm,
          rays_v, stage_v, packed_v, bitmask_v, outbuf, spmem):
    cid = lax.axis_index("c")
    sid = lax.axis_index("s")
    wid = sid * NUM_CORES + cid
    ray0 = wid * RPW

    pltpu.sync_copy(rays_hbm.at[:, pl.ds(ray0, RPW)], rays_v)

    def pack_block(k, carry):
        base_w = sid * WPS + k * PACK_CH
        pltpu.sync_copy(occs_hbm.at[:, pl.ds(base_w, PACK_CH)], stage_v)

        def pack_vec(w16, c2):
            off = w16 * CHUNK
            acc = jnp.zeros((16,), jnp.int32)
            for b in range(32):
                v = stage_v[b, pl.ds(off, 16)]
                bit = (v > 0.5).astype(jnp.int32)
                acc = acc | (bit << b)
            packed_v[pl.ds(k * PACK_CH + off, 16)] = acc
            return c2

        return lax.fori_loop(0, PACK_CH // CHUNK, pack_vec, carry)

    lax.fori_loop(0, WPS // PACK_CH, pack_block, 0)

    pltpu.sync_copy(packed_v, spmem.at[pl.ds(sid * WPS, WPS)])
    plsc.subcore_barrier()
    pltpu.sync_copy(spmem, bitmask_v)

    row_iota = lax.iota(jnp.int32, (16,))
    zero16i = jnp.zeros((16,), jnp.int32)

    def chunk_body(ch, carry):
        c16 = ch * CHUNK
        ox = rays_v[0, pl.ds(c16, 16)]
        oy = rays_v[1, pl.ds(c16, 16)]
        oz = rays_v[2, pl.ds(c16, 16)]
        dx = rays_v[3, pl.ds(c16, 16)]
        dy = rays_v[4, pl.ds(c16, 16)]
        dz = rays_v[5, pl.ds(c16, 16)]

        eps = jnp.float32(1e-8)
        dsx = jnp.where(jnp.abs(dx) < eps, eps, dx)
        dsy = jnp.where(jnp.abs(dy) < eps, eps, dy)
        dsz = jnp.where(jnp.abs(dz) < eps, eps, dz)
        t1x = (-1.0 - ox) / dsx
        t2x = (1.0 - ox) / dsx
        t1y = (-1.0 - oy) / dsy
        t2y = (1.0 - oy) / dsy
        t1z = (-1.0 - oz) / dsz
        t2z = (1.0 - oz) / dsz
        tmin = jnp.maximum(jnp.maximum(jnp.minimum(t1x, t2x),
                                       jnp.minimum(t1y, t2y)),
                           jnp.minimum(t1z, t2z))
        tmax = jnp.minimum(jnp.minimum(jnp.maximum(t1x, t2x),
                                       jnp.maximum(t1y, t2y)),
                           jnp.maximum(t1z, t2z))
        tmin = jnp.minimum(jnp.maximum(tmin, 0.0), 1e10)
        tmax = jnp.minimum(jnp.maximum(tmax, 0.0), 1e10)
        valid1 = jnp.where(tmax > tmin, 1, 0).astype(jnp.int32)
        span = tmax - tmin

        def sample_body(j, c2):
            jf = j.astype(jnp.float32)
            ts = tmin + (jf * INV_N) * span
            te = tmin + ((jf + 1.0) * INV_N) * span
            tm = 0.5 * (ts + te)
            fx = jnp.minimum(jnp.maximum((ox + tm * dx + 1.0) * 64.0, 0.0), 127.0)
            fy = jnp.minimum(jnp.maximum((oy + tm * dy + 1.0) * 64.0, 0.0), 127.0)
            fz = jnp.minimum(jnp.maximum((oz + tm * dz + 1.0) * 64.0, 0.0), 127.0)
            ix = fx.astype(jnp.int32)
            iy = fy.astype(jnp.int32)
            iz = fz.astype(jnp.int32)
            flat = ((ix << 7) | iy) << 7 | iz
            word = flat & 0xFFFF
            bpl = flat >> 16
            w = plsc.load_gather(bitmask_v, [word])
            m = ((w >> bpl) & valid1) != 0
            tsv = jnp.where(m, ts, 0.0)
            tev = jnp.where(m, te, 0.0)
            col = zero16i + 2 * j
            plsc.store_scatter(outbuf, [row_iota, col], tsv)
            plsc.store_scatter(outbuf, [row_iota, col + 1], tev)
            return c2

        lax.fori_loop(0, N_SAMPLES, sample_body, 0)
        pltpu.sync_copy(outbuf, out_hbm.at[pl.ds(ray0 + c16, 16), :])
        return carry

    lax.fori_loop(0, NCHUNK, chunk_body, 0)


@jax.jit
def kernel(rays_o, rays_d, occs):
    raysT = jnp.concatenate([rays_o.T, rays_d.T], axis=0)
    occs2d = occs.reshape(32, NWORDS)
    mesh = plsc.VectorSubcoreMesh(core_axis_name="c", subcore_axis_name="s")
    out = pl.kernel(
        _body,
        out_type=jax.ShapeDtypeStruct((N_RAYS, 2 * N_SAMPLES), jnp.float32),
        mesh=mesh,
        scratch_types=[
            pltpu.VMEM((6, RPW), jnp.float32),
            pltpu.VMEM((32, PACK_CH), jnp.float32),
            pltpu.VMEM((WPS,), jnp.int32),
            pltpu.VMEM((NWORDS,), jnp.int32),
            pltpu.VMEM((16, 2 * N_SAMPLES), jnp.float32),
            pltpu.VMEM_SHARED((NWORDS,), jnp.int32),
        ],
    )(raysT, occs2d)
    return out.reshape(N_RAYS, N_SAMPLES, 2)


# speedup vs baseline: 249.1751x; 249.1751x over previous
import jax
import jax.numpy as jnp
from jax import lax
from jax.experimental import pallas as pl
from jax.experimental.pallas import tpu as pltpu
from jax.experimental.pallas import tpu_sc as plsc

RES = 128
N_RAYS = 65536
N_SAMPLES = 128
NUM_CORES = 2
NUM_SUBCORES = 16
NW = NUM_CORES * NUM_SUBCORES
RPW = N_RAYS // NW
CHUNK = 16
NCHUNK = RPW // CHUNK
NWORDS = RES ** 3 // 32
WPS = NWORDS // NUM_SUBCORES
PACK_CH = 512
INV_N = 1.0 / N_SAMPLES


def _body(rays_hbm, occs_hbm, out_hbm,
          rays_v, stage_v, packed_v, bitmask_v, outbuf, spmem):
    cid = lax.axis_index("c")
    sid = lax.axis_index("s")
    wid = sid * NUM_CORES + cid
    ray0 = wid * RPW

    pltpu.sync_copy(rays_hbm.at[:, pl.ds(ray0, RPW)], rays_v)

    def pack_block(k, carry):
        base_w = sid * WPS + k * PACK_CH
        pltpu.sync_copy(occs_hbm.at[:, pl.ds(base_w, PACK_CH)], stage_v)

        def pack_vec(w16, c2):
            off = w16 * CHUNK
            acc = jnp.zeros((16,), jnp.int32)
            for b in range(32):
                v = stage_v[b, pl.ds(off, 16)]
                bit = (v > 0.5).astype(jnp.int32)
                acc = acc | (bit << b)
            packed_v[pl.ds(k * PACK_CH + off, 16)] = acc
            return c2

        return lax.fori_loop(0, PACK_CH // CHUNK, pack_vec, carry)

    lax.fori_loop(0, WPS // PACK_CH, pack_block, 0)

    pltpu.sync_copy(packed_v, spmem.at[pl.ds(sid * WPS, WPS)])
    plsc.subcore_barrier()
    pltpu.sync_copy(spmem, bitmask_v)

    row_iota = lax.iota(jnp.int32, 16)
    zero16i = jnp.zeros((16,), jnp.int32)

    def chunk_body(ch, carry):
        c16 = ch * CHUNK
        ox = rays_v[0, pl.ds(c16, 16)]
        oy = rays_v[1, pl.ds(c16, 16)]
        oz = rays_v[2, pl.ds(c16, 16)]
        dx = rays_v[3, pl.ds(c16, 16)]
        dy = rays_v[4, pl.ds(c16, 16)]
        dz = rays_v[5, pl.ds(c16, 16)]

        eps = jnp.float32(1e-8)
        dsx = jnp.where(jnp.abs(dx) < eps, eps, dx)
        dsy = jnp.where(jnp.abs(dy) < eps, eps, dy)
        dsz = jnp.where(jnp.abs(dz) < eps, eps, dz)
        t1x = (-1.0 - ox) / dsx
        t2x = (1.0 - ox) / dsx
        t1y = (-1.0 - oy) / dsy
        t2y = (1.0 - oy) / dsy
        t1z = (-1.0 - oz) / dsz
        t2z = (1.0 - oz) / dsz
        tmin = jnp.maximum(jnp.maximum(jnp.minimum(t1x, t2x),
                                       jnp.minimum(t1y, t2y)),
                           jnp.minimum(t1z, t2z))
        tmax = jnp.minimum(jnp.minimum(jnp.maximum(t1x, t2x),
                                       jnp.maximum(t1y, t2y)),
                           jnp.maximum(t1z, t2z))
        tmin = jnp.minimum(jnp.maximum(tmin, 0.0), 1e10)
        tmax = jnp.minimum(jnp.maximum(tmax, 0.0), 1e10)
        valid1 = jnp.where(tmax > tmin, 1, 0).astype(jnp.int32)
        span = tmax - tmin

        def sample_body(j, c2):
            jf = j.astype(jnp.float32)
            ts = tmin + (jf * INV_N) * span
            te = tmin + ((jf + 1.0) * INV_N) * span
            tm = 0.5 * (ts + te)
            fx = jnp.minimum(jnp.maximum((ox + tm * dx + 1.0) * 64.0, 0.0), 127.0)
            fy = jnp.minimum(jnp.maximum((oy + tm * dy + 1.0) * 64.0, 0.0), 127.0)
            fz = jnp.minimum(jnp.maximum((oz + tm * dz + 1.0) * 64.0, 0.0), 127.0)
            ix = fx.astype(jnp.int32)
            iy = fy.astype(jnp.int32)
            iz = fz.astype(jnp.int32)
            flat = ((ix << 7) | iy) << 7 | iz
            word = flat & 0xFFFF
            bpl = flat >> 16
            w = plsc.load_gather(bitmask_v, [word])
            m = ((w >> bpl) & valid1) != 0
            tsv = jnp.where(m, ts, 0.0)
            tev = jnp.where(m, te, 0.0)
            col = zero16i + 2 * j
            plsc.store_scatter(outbuf, [row_iota, col], tsv)
            plsc.store_scatter(outbuf, [row_iota, col + 1], tev)
            return c2

        lax.fori_loop(0, N_SAMPLES, sample_body, 0)
        pltpu.sync_copy(outbuf, out_hbm.at[pl.ds(ray0 + c16, 16), :])
        return carry

    lax.fori_loop(0, NCHUNK, chunk_body, 0)


@jax.jit
def kernel(rays_o, rays_d, occs):
    raysT = jnp.concatenate([rays_o.T, rays_d.T], axis=0)
    occs2d = occs.reshape(32, NWORDS)
    mesh = plsc.VectorSubcoreMesh(core_axis_name="c", subcore_axis_name="s")
    out = pl.kernel(
        _body,
        out_type=jax.ShapeDtypeStruct((N_RAYS, 2 * N_SAMPLES), jnp.float32),
        mesh=mesh,
        scratch_types=[
            pltpu.VMEM((6, RPW), jnp.float32),
            pltpu.VMEM((32, PACK_CH), jnp.float32),
            pltpu.VMEM((WPS,), jnp.int32),
            pltpu.VMEM((NWORDS,), jnp.int32),
            pltpu.VMEM((16, 2 * N_SAMPLES), jnp.float32),
            pltpu.VMEM_SHARED((NWORDS,), jnp.int32),
        ],
        compiler_params=pltpu.CompilerParams(needs_layout_passes=False),
    )(raysT, occs2d)
    return out.reshape(N_RAYS, N_SAMPLES, 2)
